# baseline (speedup 1.0000x reference)
import functools

import jax
import jax.numpy as jnp
from jax import lax
from jax.experimental import pallas as pl
from jax.experimental.pallas import tpu as pltpu
from jax.experimental.pallas import tpu_sc as plsc

VOCAB = 100000
HIDDEN = 128
MAXLEN = 512
BATCH = 256

NC, NS = 2, 16
NW = NC * NS
ROWS = BATCH * MAXLEN
NQ = 4
QL = MAXLEN // NQ
NB = NW // NQ
BPG = BATCH // NB
CH = QL


def _prep_body(seg_tab_ref, pos_ref, comb_ref):
    comb_ref[...] = seg_tab_ref[...][:, None, :] + pos_ref[...][None, :, :]


def _prep(segment_table, pos_weight):
    return pl.pallas_call(
        _prep_body,
        out_shape=jax.ShapeDtypeStruct((2, MAXLEN, HIDDEN), jnp.float32),
    )(segment_table, pos_weight)


@functools.partial(
    pl.kernel,
    out_type=jax.ShapeDtypeStruct((ROWS, HIDDEN), jnp.float32),
    mesh=plsc.VectorSubcoreMesh(core_axis_name="c", subcore_axis_name="s"),
    scratch_types=[
        pltpu.VMEM((BPG, CH), jnp.int32),
        pltpu.VMEM((BPG, CH), jnp.int32),
        pltpu.VMEM((2, QL, HIDDEN), jnp.float32),
        pltpu.VMEM((CH, HIDDEN), jnp.float32),
        pltpu.VMEM((CH, HIDDEN), jnp.float32),
        pltpu.SemaphoreType.DMA,
        pltpu.SemaphoreType.DMA,
        pltpu.SemaphoreType.DMA,
        pltpu.SemaphoreType.DMA,
    ],
)
def _sc_embed(tok_hbm, seg_hbm, table_hbm, comb_hbm, out_hbm,
              tki, svi, comb_l, buf_a, buf_b, sg_a, sg_b, sw_a, sw_b):
    wid = lax.axis_index("s") * NC + lax.axis_index("c")
    q = wid % NQ
    u = wid // NQ

    pltpu.sync_copy(tok_hbm.at[q, pl.ds(u * BPG, BPG)], tki)
    pltpu.sync_copy(seg_hbm.at[q, pl.ds(u * BPG, BPG)], svi)
    pltpu.sync_copy(comb_hbm.at[0, q], comb_l.at[0])
    pltpu.sync_copy(comb_hbm.at[1, q], comb_l.at[1])

    def out_at(j):
        return out_hbm.at[pl.ds((u * BPG + j) * MAXLEN + q * QL, CH)]

    def gather(j, buf, sem):
        pltpu.async_copy(table_hbm.at[tki.at[j]], buf, sem)

    def gather_wait(j, buf, sem):
        pltpu.make_async_copy(table_hbm.at[tki.at[j]], buf, sem).wait()

    def tec_add(j, buf):
        @pl.loop(0, CH // 16)
        def _g(g):
            segv = svi[j, pl.ds(g * 16, 16)]
            for l in range(16):
                s = segv[l]
                r = g * 16 + l
                for kk in range(8):
                    sl = pl.ds(kk * 16, 16)
                    plsc.addupdate(buf.at[r, sl], comb_l[s, r, sl])

    def wr(j, buf, sem):
        pltpu.async_copy(buf, out_at(j), sem)

    def wr_wait(j, buf, sem):
        pltpu.make_async_copy(buf, out_at(j), sem).wait()

    gather(0, buf_a, sg_a)

    @pl.loop(0, BPG // 2)
    def _pair(jj):
        j = jj * 2

        @pl.when(jj > 0)
        def _():
            wr_wait(j - 1, buf_b, sw_b)
        gather(j + 1, buf_b, sg_b)

        gather_wait(j, buf_a, sg_a)
        tec_add(j, buf_a)
        wr(j, buf_a, sw_a)

        gather_wait(j + 1, buf_b, sg_b)
        tec_add(j + 1, buf_b)
        wr(j + 1, buf_b, sw_b)

        wr_wait(j, buf_a, sw_a)

        @pl.when(jj < BPG // 2 - 1)
        def _():
            gather(j + 2, buf_a, sg_a)

    wr_wait(BPG - 1, buf_b, sw_b)


def kernel(tokens, segments, token_table, segment_table, pos_weight):
    comb = _prep(segment_table, pos_weight)
    comb = comb.reshape(2, NQ, QL, HIDDEN)
    tok = tokens.astype(jnp.int32).reshape(BATCH, NQ, QL).transpose(1, 0, 2)
    seg = segments.astype(jnp.int32).reshape(BATCH, NQ, QL).transpose(1, 0, 2)
    out = _sc_embed(tok, seg, token_table, comb)
    return out.reshape(BATCH, MAXLEN, HIDDEN)


# speedup vs baseline: 1.8287x; 1.8287x over previous
import functools

import jax
import jax.numpy as jnp
from jax import lax
from jax.experimental import pallas as pl
from jax.experimental.pallas import tpu as pltpu
from jax.experimental.pallas import tpu_sc as plsc

VOCAB = 100000
HIDDEN = 128
MAXLEN = 512
BATCH = 256

NC, NS = 2, 16
NW = NC * NS
ROWS = BATCH * MAXLEN
RPW = ROWS // NW
CH = 128
NCHUNK = RPW // CH
CROWS = 2 * MAXLEN
CPW = CROWS // NS


@functools.partial(
    pl.kernel,
    out_type=jax.ShapeDtypeStruct((ROWS, HIDDEN), jnp.float32),
    mesh=plsc.VectorSubcoreMesh(core_axis_name="c", subcore_axis_name="s"),
    scratch_types=[
        pltpu.VMEM((NCHUNK, CH), jnp.int32),
        pltpu.VMEM((NCHUNK, CH), jnp.int32),
        pltpu.VMEM((NCHUNK, CH), jnp.int32),
        pltpu.VMEM((CPW, HIDDEN), jnp.float32),
        pltpu.VMEM((1, HIDDEN), jnp.float32),
        pltpu.VMEM((CPW, HIDDEN), jnp.float32),
        pltpu.VMEM((CH, HIDDEN), jnp.float32),
        pltpu.VMEM((CH, HIDDEN), jnp.float32),
        pltpu.VMEM_SHARED((CROWS, HIDDEN), jnp.float32),
        pltpu.SemaphoreType.DMA,
        pltpu.SemaphoreType.DMA,
        pltpu.SemaphoreType.DMA,
        pltpu.SemaphoreType.DMA,
    ],
)
def _sc_embed(tok_hbm, seg_hbm, table_hbm, seg_tab_hbm, pos_hbm, out_hbm,
              tki, svi, cvi, pos_l, seg_l, comb_tmp, buf_a, buf_b, comb_sp,
              sg_a, sg_b, sw_a, sw_b):
    cid = lax.axis_index("c")
    sid = lax.axis_index("s")
    wid = sid * NC + cid
    base = wid * RPW

    s_blk = sid // (NS // 2)
    l0 = (sid % (NS // 2)) * CPW
    pltpu.sync_copy(pos_hbm.at[pl.ds(l0, CPW)], pos_l)
    pltpu.sync_copy(seg_tab_hbm.at[pl.ds(s_blk, 1)], seg_l)
    pltpu.sync_copy(tok_hbm.at[wid], tki)
    pltpu.sync_copy(seg_hbm.at[wid], svi)

    @pl.loop(0, CPW)
    def _crow(r):
        for kk in range(HIDDEN // 16):
            sl = pl.ds(kk * 16, 16)
            comb_tmp[r, sl] = pos_l[r, sl] + seg_l[0, sl]

    pltpu.sync_copy(comb_tmp, comb_sp.at[pl.ds(sid * CPW, CPW)])

    lane = lax.iota(jnp.int32, 16)

    @pl.loop(0, NCHUNK)
    def _cidx(j):
        pos0 = (j % (MAXLEN // CH)) * CH
        for g in range(CH // 16):
            sl = pl.ds(g * 16, 16)
            cvi[j, sl] = svi[j, sl] * MAXLEN + (pos0 + g * 16) + lane

    plsc.subcore_barrier()

    def out_at(j):
        return out_hbm.at[pl.ds(base + j * CH, CH)]

    def g_init(j, buf, sem):
        pltpu.async_copy(comb_sp.at[cvi.at[j]], buf, sem)

    def g_init_wait(j, buf, sem):
        pltpu.make_async_copy(comb_sp.at[cvi.at[j]], buf, sem).wait()

    def g_add(j, buf, sem):
        pltpu.async_copy(table_hbm.at[tki.at[j]], buf, sem, add=True)

    def g_add_wait(j, buf, sem):
        pltpu.make_async_copy(table_hbm.at[tki.at[j]], buf, sem).wait()

    def wr(j, buf, sem):
        pltpu.async_copy(buf, out_at(j), sem)

    def wr_wait(j, buf, sem):
        pltpu.make_async_copy(buf, out_at(j), sem).wait()

    g_init(0, buf_a, sg_a)

    @pl.loop(0, NCHUNK // 2)
    def _pair(jj):
        j = jj * 2

        @pl.when(jj > 0)
        def _():
            wr_wait(j - 1, buf_b, sw_b)
        g_init(j + 1, buf_b, sg_b)

        g_init_wait(j, buf_a, sg_a)
        g_add(j, buf_a, sg_a)
        g_add_wait(j, buf_a, sg_a)
        wr(j, buf_a, sw_a)

        g_init_wait(j + 1, buf_b, sg_b)
        g_add(j + 1, buf_b, sg_b)
        g_add_wait(j + 1, buf_b, sg_b)
        wr(j + 1, buf_b, sw_b)

        wr_wait(j, buf_a, sw_a)

        @pl.when(jj < NCHUNK // 2 - 1)
        def _():
            g_init(j + 2, buf_a, sg_a)

    wr_wait(NCHUNK - 1, buf_b, sw_b)


def kernel(tokens, segments, token_table, segment_table, pos_weight):
    tok = tokens.astype(jnp.int32).reshape(NW, NCHUNK, CH)
    seg = segments.astype(jnp.int32).reshape(NW, NCHUNK, CH)
    out = _sc_embed(tok, seg, token_table, segment_table, pos_weight)
    return out.reshape(BATCH, MAXLEN, HIDDEN)
